# baseline (speedup 1.0000x reference)
import functools

import jax
import jax.numpy as jnp
from jax import lax
from jax.experimental import pallas as pl
from jax.experimental.pallas import tpu as pltpu
from jax.experimental.pallas import tpu_sc as plsc

N_ROWS = 524288
N_CLS = 100
EPS_SPEC = 1e-07

BLK = 2048
GRID = N_ROWS // BLK

NC, NS, LANES = 2, 16, 16
NW = NC * NS
CHUNK = N_ROWS // NW
NVEC = CHUNK // LANES
CPAD = 128
HSECT = LANES * CPAD
HWORDS = 3 * HSECT


def _argmax_body(yp_ref, out_ref):
    yp = yp_ref[...]
    m = jnp.max(yp, axis=1, keepdims=True)
    colid = lax.broadcasted_iota(jnp.int32, (BLK, N_CLS), 1)
    cand = jnp.where(yp == m, colid, N_CLS)
    pred2 = jnp.min(cand, axis=1, keepdims=True)
    onehot = (colid == pred2).astype(jnp.bfloat16)
    iota_row = lax.broadcasted_iota(jnp.int32, (1, N_CLS), 1).astype(jnp.bfloat16)
    pred_t = lax.dot_general(iota_row, onehot, (((1,), (1,)), ((), ())),
                             preferred_element_type=jnp.float32)
    out_ref[...] = pred_t.astype(jnp.int32).reshape(1, 1, BLK)


_argmax_call = pl.pallas_call(
    _argmax_body,
    grid=(GRID,),
    in_specs=[pl.BlockSpec((BLK, N_CLS), lambda i: (i, 0))],
    out_specs=pl.BlockSpec((1, 1, BLK), lambda i: (i, 0, 0)),
    out_shape=jax.ShapeDtypeStruct((GRID, 1, BLK), jnp.int32),
)


def _hist_body(preds_hbm, yt_hbm, out_hbm, pv, tv, hist, idxv, shared, sem):
    cid = lax.axis_index("c")
    sid = lax.axis_index("s")
    wid = cid * NS + sid
    base = wid * CHUNK

    pltpu.sync_copy(preds_hbm.at[pl.ds(base, CHUNK)], pv)
    pltpu.sync_copy(yt_hbm.at[pl.ds(base, CHUNK)], tv)

    zeros16 = jnp.zeros((LANES,), jnp.float32)

    def zero_body(j, _):
        for k in range(CPAD // LANES):
            hist[j, pl.ds(k * LANES, LANES)] = zeros16
        return 0

    lax.fori_loop(0, CPAD, zero_body, 0)

    lane = jnp.arange(LANES, dtype=jnp.int32)
    ones16 = jnp.ones((LANES,), jnp.float32)

    def acc_body(j, _):
        p = pv[pl.ds(j * LANES, LANES)]
        t = tv[pl.ds(j * LANES, LANES)]
        plsc.addupdate_scatter(hist, [p, lane], ones16)
        plsc.addupdate_scatter(hist, [t, lane + LANES], ones16)
        hit = jnp.where(p == t, 1.0, 0.0).astype(jnp.float32)
        plsc.addupdate_scatter(hist, [p, lane + 2 * LANES], hit)
        return 0

    lax.fori_loop(0, NVEC, acc_body, 0)

    def idx_body(j, _):
        idxv[pl.ds(j * LANES, LANES)] = lane + j * LANES
        return 0

    lax.fori_loop(0, CPAD // LANES, idx_body, 0)

    @pl.when(sid == 0)
    def _seed():
        pltpu.sync_copy(hist, shared)

    plsc.subcore_barrier()

    @pl.when(sid != 0)
    def _accum():
        pltpu.sync_copy(hist, shared.at[idxv], add=True)

    plsc.subcore_barrier()

    @pl.when(sid == 0)
    def _emit():
        pltpu.sync_copy(shared, out_hbm.at[cid])


@functools.cache
def _hist_call():
    return functools.partial(
        pl.kernel,
        out_type=jax.ShapeDtypeStruct((NC, CPAD, CPAD), jnp.float32),
        mesh=plsc.VectorSubcoreMesh(core_axis_name="c", subcore_axis_name="s",
                                    num_cores=NC, num_subcores=NS),
        scratch_types=[
            pltpu.VMEM((CHUNK,), jnp.int32),
            pltpu.VMEM((CHUNK,), jnp.int32),
            pltpu.VMEM((CPAD, CPAD), jnp.float32),
            pltpu.VMEM((CPAD,), jnp.int32),
            pltpu.VMEM_SHARED((CPAD, CPAD), jnp.float32),
            pltpu.SemaphoreType.DMA,
        ],
        compiler_params=pltpu.CompilerParams(needs_layout_passes=False),
    )(_hist_body)


def _finalize_body(part_ref, out_ref):
    s = part_ref[0] + part_ref[1]
    col = jnp.sum(s[:, 0:LANES], axis=1, keepdims=True)
    row = jnp.sum(s[:, LANES:2 * LANES], axis=1, keepdims=True)
    diag = jnp.sum(s[:, 2 * LANES:3 * LANES], axis=1, keepdims=True)
    total = jnp.float32(N_ROWS)
    tn = total - row - col + diag
    fp = row - diag
    ratio = tn / (tn + fp + jnp.float32(EPS_SPEC))
    classid = lax.broadcasted_iota(jnp.int32, (CPAD, 1), 0)
    ratio = jnp.where(classid < N_CLS, ratio, 0.0)
    out_ref[...] = lax.broadcast(jnp.sum(ratio) / N_CLS, (1, 1))


_finalize_call = pl.pallas_call(
    _finalize_body,
    out_shape=jax.ShapeDtypeStruct((1, 1), jnp.float32),
)


@jax.jit
def kernel(y_pred, y_true):
    preds = _argmax_call(y_pred).reshape(N_ROWS)
    partials = _hist_call()(preds, y_true)
    return _finalize_call(partials)[0, 0]


# speedup vs baseline: 1.5711x; 1.5711x over previous
import functools

import jax
import jax.numpy as jnp
from jax import lax
from jax.experimental import pallas as pl
from jax.experimental.pallas import tpu as pltpu
from jax.experimental.pallas import tpu_sc as plsc

N_ROWS = 524288
N_CLS = 100
EPS_SPEC = 1e-07

BLK = 4096
GRID = N_ROWS // BLK

NC, NS, LANES = 2, 16, 16
NW = NC * NS
CHUNK = N_ROWS // NW
NVEC = CHUNK // LANES
CPAD = 128
HSECT = LANES * CPAD
HWORDS = 3 * HSECT


def _argmax_body(yp_ref, out_ref):
    yp = yp_ref[...]
    m = jnp.max(yp, axis=1, keepdims=True)
    colid_f = lax.broadcasted_iota(jnp.int32, (BLK, N_CLS), 1).astype(
        jnp.float32)
    cand = jnp.where(yp == m, colid_f, jnp.float32(N_CLS))
    pred2 = jnp.min(cand, axis=1, keepdims=True)
    onehot = (colid_f == pred2).astype(jnp.bfloat16)
    iota_row = lax.broadcasted_iota(jnp.int32, (1, N_CLS), 1).astype(
        jnp.bfloat16)
    pred_t = lax.dot_general(iota_row, onehot, (((1,), (1,)), ((), ())),
                             preferred_element_type=jnp.float32)
    out_ref[...] = pred_t.astype(jnp.int32).reshape(1, 1, BLK)


_argmax_call = pl.pallas_call(
    _argmax_body,
    grid=(GRID,),
    in_specs=[pl.BlockSpec((BLK, N_CLS), lambda i: (i, 0))],
    out_specs=pl.BlockSpec((1, 1, BLK), lambda i: (i, 0, 0)),
    out_shape=jax.ShapeDtypeStruct((GRID, 1, BLK), jnp.int32),
)


def _hist_body(preds_hbm, yt_hbm, out_hbm, pv, tv, hist, idxv, shared, sem):
    cid = lax.axis_index("c")
    sid = lax.axis_index("s")
    wid = cid * NS + sid
    base = wid * CHUNK

    pltpu.sync_copy(preds_hbm.at[pl.ds(base, CHUNK)], pv)
    pltpu.sync_copy(yt_hbm.at[pl.ds(base, CHUNK)], tv)

    zeros16 = jnp.zeros((LANES,), jnp.float32)

    def zero_body(j, _):
        for k in range(CPAD // LANES):
            hist[j, pl.ds(k * LANES, LANES)] = zeros16
        return 0

    lax.fori_loop(0, CPAD, zero_body, 0)

    lane = jnp.arange(LANES, dtype=jnp.int32)
    ones16 = jnp.ones((LANES,), jnp.float32)

    def acc_body(j, _):
        p = pv[pl.ds(j * LANES, LANES)]
        t = tv[pl.ds(j * LANES, LANES)]
        plsc.addupdate_scatter(hist, [p, lane], ones16)
        plsc.addupdate_scatter(hist, [t, lane + LANES], ones16)
        hit = jnp.where(p == t, 1.0, 0.0).astype(jnp.float32)
        plsc.addupdate_scatter(hist, [p, lane + 2 * LANES], hit)
        return 0

    lax.fori_loop(0, NVEC, acc_body, 0)

    def idx_body(j, _):
        idxv[pl.ds(j * LANES, LANES)] = lane + j * LANES
        return 0

    lax.fori_loop(0, CPAD // LANES, idx_body, 0)

    @pl.when(sid == 0)
    def _seed():
        pltpu.sync_copy(hist, shared)

    plsc.subcore_barrier()

    @pl.when(sid != 0)
    def _accum():
        pltpu.sync_copy(hist, shared.at[idxv], add=True)

    plsc.subcore_barrier()

    @pl.when(sid == 0)
    def _emit():
        pltpu.sync_copy(shared, out_hbm.at[cid])


@functools.cache
def _hist_call():
    return functools.partial(
        pl.kernel,
        out_type=jax.ShapeDtypeStruct((NC, CPAD, CPAD), jnp.float32),
        mesh=plsc.VectorSubcoreMesh(core_axis_name="c", subcore_axis_name="s",
                                    num_cores=NC, num_subcores=NS),
        scratch_types=[
            pltpu.VMEM((CHUNK,), jnp.int32),
            pltpu.VMEM((CHUNK,), jnp.int32),
            pltpu.VMEM((CPAD, CPAD), jnp.float32),
            pltpu.VMEM((CPAD,), jnp.int32),
            pltpu.VMEM_SHARED((CPAD, CPAD), jnp.float32),
            pltpu.SemaphoreType.DMA,
        ],
        compiler_params=pltpu.CompilerParams(needs_layout_passes=False),
    )(_hist_body)


def _finalize_body(part_ref, out_ref):
    s = part_ref[0] + part_ref[1]
    col = jnp.sum(s[:, 0:LANES], axis=1, keepdims=True)
    row = jnp.sum(s[:, LANES:2 * LANES], axis=1, keepdims=True)
    diag = jnp.sum(s[:, 2 * LANES:3 * LANES], axis=1, keepdims=True)
    total = jnp.float32(N_ROWS)
    tn = total - row - col + diag
    fp = row - diag
    ratio = tn / (tn + fp + jnp.float32(EPS_SPEC))
    classid = lax.broadcasted_iota(jnp.int32, (CPAD, 1), 0)
    ratio = jnp.where(classid < N_CLS, ratio, 0.0)
    out_ref[...] = lax.broadcast(jnp.sum(ratio) / N_CLS, (1, 1))


_finalize_call = pl.pallas_call(
    _finalize_body,
    out_shape=jax.ShapeDtypeStruct((1, 1), jnp.float32),
)


@jax.jit
def kernel(y_pred, y_true):
    preds = _argmax_call(y_pred).reshape(N_ROWS)
    partials = _hist_call()(preds, y_true)
    return _finalize_call(partials)[0, 0]


# speedup vs baseline: 4.4148x; 2.8100x over previous
import functools

import jax
import jax.numpy as jnp
from jax import lax
from jax.experimental import pallas as pl
from jax.experimental.pallas import tpu as pltpu
from jax.experimental.pallas import tpu_sc as plsc

N_ROWS = 524288
N_CLS = 100
EPS_SPEC = 1e-07

BLK = 4096
GRID = N_ROWS // BLK

NC, NS, LANES = 2, 16, 16
NW = NC * NS
CHUNK = N_ROWS // NW
NVEC = CHUNK // LANES
CPAD = 128
HSECT = LANES * CPAD
HWORDS = 3 * HSECT


def _argmax_body(ypt_ref, out_ref):
    yp = ypt_ref[...]
    m = jnp.max(yp, axis=0, keepdims=True)
    rowid_f = lax.broadcasted_iota(jnp.int32, (N_CLS, BLK), 0).astype(
        jnp.float32)
    cand = jnp.where(yp == m, rowid_f, jnp.float32(N_CLS))
    pred = jnp.min(cand, axis=0, keepdims=True)
    out_ref[...] = pred.astype(jnp.int32)


_argmax_call = pl.pallas_call(
    _argmax_body,
    grid=(GRID,),
    in_specs=[pl.BlockSpec((N_CLS, BLK), lambda i: (0, i))],
    out_specs=pl.BlockSpec((1, BLK), lambda i: (0, i)),
    out_shape=jax.ShapeDtypeStruct((1, N_ROWS), jnp.int32),
)


def _hist_body(preds_hbm, yt_hbm, out_hbm, pv, tv, hist, idxv, shared, sem):
    cid = lax.axis_index("c")
    sid = lax.axis_index("s")
    wid = cid * NS + sid
    base = wid * CHUNK

    pltpu.sync_copy(preds_hbm.at[pl.ds(base, CHUNK)], pv)
    pltpu.sync_copy(yt_hbm.at[pl.ds(base, CHUNK)], tv)

    zeros16 = jnp.zeros((LANES,), jnp.float32)

    def zero_body(j, _):
        for k in range(CPAD // LANES):
            hist[j, pl.ds(k * LANES, LANES)] = zeros16
        return 0

    lax.fori_loop(0, CPAD, zero_body, 0)

    lane = jnp.arange(LANES, dtype=jnp.int32)
    ones16 = jnp.ones((LANES,), jnp.float32)

    def acc_body(j, _):
        p = pv[pl.ds(j * LANES, LANES)]
        t = tv[pl.ds(j * LANES, LANES)]
        plsc.addupdate_scatter(hist, [p, lane], ones16)
        plsc.addupdate_scatter(hist, [t, lane + LANES], ones16)
        hit = jnp.where(p == t, 1.0, 0.0).astype(jnp.float32)
        plsc.addupdate_scatter(hist, [p, lane + 2 * LANES], hit)
        return 0

    lax.fori_loop(0, NVEC, acc_body, 0)

    def idx_body(j, _):
        idxv[pl.ds(j * LANES, LANES)] = lane + j * LANES
        return 0

    lax.fori_loop(0, CPAD // LANES, idx_body, 0)

    @pl.when(sid == 0)
    def _seed():
        pltpu.sync_copy(hist, shared)

    plsc.subcore_barrier()

    @pl.when(sid != 0)
    def _accum():
        pltpu.sync_copy(hist, shared.at[idxv], add=True)

    plsc.subcore_barrier()

    @pl.when(sid == 0)
    def _emit():
        pltpu.sync_copy(shared, out_hbm.at[cid])


@functools.cache
def _hist_call():
    return functools.partial(
        pl.kernel,
        out_type=jax.ShapeDtypeStruct((NC, CPAD, CPAD), jnp.float32),
        mesh=plsc.VectorSubcoreMesh(core_axis_name="c", subcore_axis_name="s",
                                    num_cores=NC, num_subcores=NS),
        scratch_types=[
            pltpu.VMEM((CHUNK,), jnp.int32),
            pltpu.VMEM((CHUNK,), jnp.int32),
            pltpu.VMEM((CPAD, CPAD), jnp.float32),
            pltpu.VMEM((CPAD,), jnp.int32),
            pltpu.VMEM_SHARED((CPAD, CPAD), jnp.float32),
            pltpu.SemaphoreType.DMA,
        ],
        compiler_params=pltpu.CompilerParams(needs_layout_passes=False),
    )(_hist_body)


def _finalize_body(part_ref, out_ref):
    s = part_ref[0] + part_ref[1]
    col = jnp.sum(s[:, 0:LANES], axis=1, keepdims=True)
    row = jnp.sum(s[:, LANES:2 * LANES], axis=1, keepdims=True)
    diag = jnp.sum(s[:, 2 * LANES:3 * LANES], axis=1, keepdims=True)
    total = jnp.float32(N_ROWS)
    tn = total - row - col + diag
    fp = row - diag
    ratio = tn / (tn + fp + jnp.float32(EPS_SPEC))
    classid = lax.broadcasted_iota(jnp.int32, (CPAD, 1), 0)
    ratio = jnp.where(classid < N_CLS, ratio, 0.0)
    out_ref[...] = lax.broadcast(jnp.sum(ratio) / N_CLS, (1, 1))


_finalize_call = pl.pallas_call(
    _finalize_body,
    out_shape=jax.ShapeDtypeStruct((1, 1), jnp.float32),
)


@jax.jit
def kernel(y_pred, y_true):
    preds = _argmax_call(y_pred.T).reshape(N_ROWS)
    partials = _hist_call()(preds, y_true)
    return _finalize_call(partials)[0, 0]


# speedup vs baseline: 5.5615x; 1.2598x over previous
import functools

import jax
import jax.numpy as jnp
from jax import lax
from jax.experimental import pallas as pl
from jax.experimental.pallas import tpu as pltpu
from jax.experimental.pallas import tpu_sc as plsc

N_ROWS = 524288
N_CLS = 100
EPS_SPEC = 1e-07

BLK = 8192
GRID = N_ROWS // BLK

NC, NS, LANES = 2, 16, 16
NW = NC * NS
CHUNK = N_ROWS // NW
NVEC = CHUNK // LANES
CPAD = 128
HSECT = LANES * CPAD
HWORDS = 3 * HSECT


def _argmax_body(ypt_ref, out_ref):
    yp = ypt_ref[...]
    m = jnp.max(yp, axis=0, keepdims=True)
    rowid_f = lax.broadcasted_iota(jnp.int32, (N_CLS, BLK), 0).astype(
        jnp.float32)
    cand = jnp.where(yp == m, rowid_f, jnp.float32(N_CLS))
    pred = jnp.min(cand, axis=0, keepdims=True)
    out_ref[...] = pred.astype(jnp.int32)


_argmax_call = pl.pallas_call(
    _argmax_body,
    grid=(GRID,),
    in_specs=[pl.BlockSpec((N_CLS, BLK), lambda i: (0, i))],
    out_specs=pl.BlockSpec((1, BLK), lambda i: (0, i)),
    out_shape=jax.ShapeDtypeStruct((1, N_ROWS), jnp.int32),
)


def _hist_body(preds_hbm, yt_hbm, out_hbm, pv, tv, hist, idxv, shared, sem):
    cid = lax.axis_index("c")
    sid = lax.axis_index("s")
    wid = cid * NS + sid
    base = wid * CHUNK

    pltpu.sync_copy(preds_hbm.at[pl.ds(base, CHUNK)], pv)
    pltpu.sync_copy(yt_hbm.at[pl.ds(base, CHUNK)], tv)

    zeros16 = jnp.zeros((LANES,), jnp.float32)

    def zero_body(j, _):
        for k in range(CPAD // LANES):
            hist[j, pl.ds(k * LANES, LANES)] = zeros16
        return 0

    lax.fori_loop(0, CPAD, zero_body, 0)

    lane = jnp.arange(LANES, dtype=jnp.int32)
    ones16 = jnp.ones((LANES,), jnp.float32)

    def acc_body(j, _):
        p = pv[pl.ds(j * LANES, LANES)]
        t = tv[pl.ds(j * LANES, LANES)]
        plsc.addupdate_scatter(hist, [p, lane], ones16)
        plsc.addupdate_scatter(hist, [t, lane + LANES], ones16)
        hit = jnp.where(p == t, 1.0, 0.0).astype(jnp.float32)
        plsc.addupdate_scatter(hist, [p, lane + 2 * LANES], hit)
        return 0

    lax.fori_loop(0, NVEC, acc_body, 0)

    def idx_body(j, _):
        idxv[pl.ds(j * LANES, LANES)] = lane + j * LANES
        return 0

    lax.fori_loop(0, CPAD // LANES, idx_body, 0)

    @pl.when(sid == 0)
    def _seed():
        pltpu.sync_copy(hist, shared)

    plsc.subcore_barrier()

    @pl.when(sid != 0)
    def _accum():
        pltpu.sync_copy(hist, shared.at[idxv], add=True)

    plsc.subcore_barrier()

    @pl.when(sid == 0)
    def _emit():
        pltpu.sync_copy(shared, out_hbm.at[cid])


@functools.cache
def _hist_call():
    return functools.partial(
        pl.kernel,
        out_type=jax.ShapeDtypeStruct((NC, CPAD, CPAD), jnp.float32),
        mesh=plsc.VectorSubcoreMesh(core_axis_name="c", subcore_axis_name="s",
                                    num_cores=NC, num_subcores=NS),
        scratch_types=[
            pltpu.VMEM((CHUNK,), jnp.int32),
            pltpu.VMEM((CHUNK,), jnp.int32),
            pltpu.VMEM((CPAD, CPAD), jnp.float32),
            pltpu.VMEM((CPAD,), jnp.int32),
            pltpu.VMEM_SHARED((CPAD, CPAD), jnp.float32),
            pltpu.SemaphoreType.DMA,
        ],
        compiler_params=pltpu.CompilerParams(needs_layout_passes=False),
    )(_hist_body)


def _finalize_body(part_ref, out_ref):
    s = part_ref[0] + part_ref[1]
    col = jnp.sum(s[:, 0:LANES], axis=1, keepdims=True)
    row = jnp.sum(s[:, LANES:2 * LANES], axis=1, keepdims=True)
    diag = jnp.sum(s[:, 2 * LANES:3 * LANES], axis=1, keepdims=True)
    total = jnp.float32(N_ROWS)
    tn = total - row - col + diag
    fp = row - diag
    ratio = tn / (tn + fp + jnp.float32(EPS_SPEC))
    classid = lax.broadcasted_iota(jnp.int32, (CPAD, 1), 0)
    ratio = jnp.where(classid < N_CLS, ratio, 0.0)
    out_ref[...] = lax.broadcast(jnp.sum(ratio) / N_CLS, (1, 1))


_finalize_call = pl.pallas_call(
    _finalize_body,
    out_shape=jax.ShapeDtypeStruct((1, 1), jnp.float32),
)


@jax.jit
def kernel(y_pred, y_true):
    preds = _argmax_call(y_pred.T).reshape(N_ROWS)
    partials = _hist_call()(preds, y_true)
    return _finalize_call(partials)[0, 0]


# speedup vs baseline: 6.4259x; 1.1554x over previous
import functools

import jax
import jax.numpy as jnp
from jax import lax
from jax.experimental import pallas as pl
from jax.experimental.pallas import tpu as pltpu
from jax.experimental.pallas import tpu_sc as plsc

N_ROWS = 524288
N_CLS = 100
EPS_SPEC = 1e-07

BLK = 16384
GRID = N_ROWS // BLK

NC, NS, LANES = 2, 16, 16
NW = NC * NS
CHUNK = N_ROWS // NW
NVEC = CHUNK // LANES
CPAD = 128
HSECT = LANES * CPAD
HWORDS = 3 * HSECT


def _argmax_body(ypt_ref, out_ref):
    yp = ypt_ref[...]
    m = jnp.max(yp, axis=0, keepdims=True)
    rowid_f = lax.broadcasted_iota(jnp.int32, (N_CLS, BLK), 0).astype(
        jnp.float32)
    cand = jnp.where(yp == m, rowid_f, jnp.float32(N_CLS))
    pred = jnp.min(cand, axis=0, keepdims=True)
    out_ref[...] = pred.astype(jnp.int32)


_argmax_call = pl.pallas_call(
    _argmax_body,
    grid=(GRID,),
    in_specs=[pl.BlockSpec((N_CLS, BLK), lambda i: (0, i))],
    out_specs=pl.BlockSpec((1, BLK), lambda i: (0, i)),
    out_shape=jax.ShapeDtypeStruct((1, N_ROWS), jnp.int32),
)


def _hist_body(preds_hbm, yt_hbm, out_hbm, pv, tv, hist, idxv, shared, sem):
    cid = lax.axis_index("c")
    sid = lax.axis_index("s")
    wid = cid * NS + sid
    base = wid * CHUNK

    pltpu.sync_copy(preds_hbm.at[pl.ds(base, CHUNK)], pv)
    pltpu.sync_copy(yt_hbm.at[pl.ds(base, CHUNK)], tv)

    zeros16 = jnp.zeros((LANES,), jnp.float32)

    def zero_body(j, _):
        for k in range(CPAD // LANES):
            hist[j, pl.ds(k * LANES, LANES)] = zeros16
        return 0

    lax.fori_loop(0, CPAD, zero_body, 0)

    lane = jnp.arange(LANES, dtype=jnp.int32)
    ones16 = jnp.ones((LANES,), jnp.float32)

    def acc_body(j, _):
        p = pv[pl.ds(j * LANES, LANES)]
        t = tv[pl.ds(j * LANES, LANES)]
        plsc.addupdate_scatter(hist, [p, lane], ones16)
        plsc.addupdate_scatter(hist, [t, lane + LANES], ones16)
        hit = jnp.where(p == t, 1.0, 0.0).astype(jnp.float32)
        plsc.addupdate_scatter(hist, [p, lane + 2 * LANES], hit)
        return 0

    lax.fori_loop(0, NVEC, acc_body, 0)

    def idx_body(j, _):
        idxv[pl.ds(j * LANES, LANES)] = lane + j * LANES
        return 0

    lax.fori_loop(0, CPAD // LANES, idx_body, 0)

    @pl.when(sid == 0)
    def _seed():
        pltpu.sync_copy(hist, shared)

    plsc.subcore_barrier()

    @pl.when(sid != 0)
    def _accum():
        pltpu.sync_copy(hist, shared.at[idxv], add=True)

    plsc.subcore_barrier()

    @pl.when(sid == 0)
    def _emit():
        pltpu.sync_copy(shared, out_hbm.at[cid])


@functools.cache
def _hist_call():
    return functools.partial(
        pl.kernel,
        out_type=jax.ShapeDtypeStruct((NC, CPAD, CPAD), jnp.float32),
        mesh=plsc.VectorSubcoreMesh(core_axis_name="c", subcore_axis_name="s",
                                    num_cores=NC, num_subcores=NS),
        scratch_types=[
            pltpu.VMEM((CHUNK,), jnp.int32),
            pltpu.VMEM((CHUNK,), jnp.int32),
            pltpu.VMEM((CPAD, CPAD), jnp.float32),
            pltpu.VMEM((CPAD,), jnp.int32),
            pltpu.VMEM_SHARED((CPAD, CPAD), jnp.float32),
            pltpu.SemaphoreType.DMA,
        ],
        compiler_params=pltpu.CompilerParams(needs_layout_passes=False),
    )(_hist_body)


def _finalize_body(part_ref, out_ref):
    s = part_ref[0] + part_ref[1]
    col = jnp.sum(s[:, 0:LANES], axis=1, keepdims=True)
    row = jnp.sum(s[:, LANES:2 * LANES], axis=1, keepdims=True)
    diag = jnp.sum(s[:, 2 * LANES:3 * LANES], axis=1, keepdims=True)
    total = jnp.float32(N_ROWS)
    tn = total - row - col + diag
    fp = row - diag
    ratio = tn / (tn + fp + jnp.float32(EPS_SPEC))
    classid = lax.broadcasted_iota(jnp.int32, (CPAD, 1), 0)
    ratio = jnp.where(classid < N_CLS, ratio, 0.0)
    out_ref[...] = lax.broadcast(jnp.sum(ratio) / N_CLS, (1, 1))


_finalize_call = pl.pallas_call(
    _finalize_body,
    out_shape=jax.ShapeDtypeStruct((1, 1), jnp.float32),
)


@jax.jit
def kernel(y_pred, y_true):
    preds = _argmax_call(y_pred.T).reshape(N_ROWS)
    partials = _hist_call()(preds, y_true)
    return _finalize_call(partials)[0, 0]


# speedup vs baseline: 6.7550x; 1.0512x over previous
import functools

import jax
import jax.numpy as jnp
from jax import lax
from jax.experimental import pallas as pl
from jax.experimental.pallas import tpu as pltpu
from jax.experimental.pallas import tpu_sc as plsc

N_ROWS = 524288
N_CLS = 100
EPS_SPEC = 1e-07

BLK = 32768
GRID = N_ROWS // BLK

NC, NS, LANES = 2, 16, 16
NW = NC * NS
CHUNK = N_ROWS // NW
NVEC = CHUNK // LANES
CPAD = 128
HSECT = LANES * CPAD
HWORDS = 3 * HSECT


def _argmax_body(ypt_ref, out_ref):
    yp = ypt_ref[...]
    m = jnp.max(yp, axis=0, keepdims=True)
    rowid_f = lax.broadcasted_iota(jnp.int32, (N_CLS, BLK), 0).astype(
        jnp.float32)
    cand = jnp.where(yp == m, rowid_f, jnp.float32(N_CLS))
    pred = jnp.min(cand, axis=0, keepdims=True)
    out_ref[...] = pred.astype(jnp.int32)


_argmax_call = pl.pallas_call(
    _argmax_body,
    grid=(GRID,),
    in_specs=[pl.BlockSpec((N_CLS, BLK), lambda i: (0, i))],
    out_specs=pl.BlockSpec((1, BLK), lambda i: (0, i)),
    out_shape=jax.ShapeDtypeStruct((1, N_ROWS), jnp.int32),
)


def _hist_body(preds_hbm, yt_hbm, out_hbm, pv, tv, hist, idxv, shared, sem):
    cid = lax.axis_index("c")
    sid = lax.axis_index("s")
    wid = cid * NS + sid
    base = wid * CHUNK

    pltpu.sync_copy(preds_hbm.at[pl.ds(base, CHUNK)], pv)
    pltpu.sync_copy(yt_hbm.at[pl.ds(base, CHUNK)], tv)

    zeros16 = jnp.zeros((LANES,), jnp.float32)

    def zero_body(j, _):
        for k in range(CPAD // LANES):
            hist[j, pl.ds(k * LANES, LANES)] = zeros16
        return 0

    lax.fori_loop(0, CPAD, zero_body, 0)

    lane = jnp.arange(LANES, dtype=jnp.int32)
    ones16 = jnp.ones((LANES,), jnp.float32)

    def acc_body(j, _):
        p = pv[pl.ds(j * LANES, LANES)]
        t = tv[pl.ds(j * LANES, LANES)]
        plsc.addupdate_scatter(hist, [p, lane], ones16)
        plsc.addupdate_scatter(hist, [t, lane + LANES], ones16)
        hit = jnp.where(p == t, 1.0, 0.0).astype(jnp.float32)
        plsc.addupdate_scatter(hist, [p, lane + 2 * LANES], hit)
        return 0

    lax.fori_loop(0, NVEC, acc_body, 0)

    def idx_body(j, _):
        idxv[pl.ds(j * LANES, LANES)] = lane + j * LANES
        return 0

    lax.fori_loop(0, CPAD // LANES, idx_body, 0)

    @pl.when(sid == 0)
    def _seed():
        pltpu.sync_copy(hist, shared)

    plsc.subcore_barrier()

    @pl.when(sid != 0)
    def _accum():
        pltpu.sync_copy(hist, shared.at[idxv], add=True)

    plsc.subcore_barrier()

    @pl.when(sid == 0)
    def _emit():
        pltpu.sync_copy(shared, out_hbm.at[cid])


@functools.cache
def _hist_call():
    return functools.partial(
        pl.kernel,
        out_type=jax.ShapeDtypeStruct((NC, CPAD, CPAD), jnp.float32),
        mesh=plsc.VectorSubcoreMesh(core_axis_name="c", subcore_axis_name="s",
                                    num_cores=NC, num_subcores=NS),
        scratch_types=[
            pltpu.VMEM((CHUNK,), jnp.int32),
            pltpu.VMEM((CHUNK,), jnp.int32),
            pltpu.VMEM((CPAD, CPAD), jnp.float32),
            pltpu.VMEM((CPAD,), jnp.int32),
            pltpu.VMEM_SHARED((CPAD, CPAD), jnp.float32),
            pltpu.SemaphoreType.DMA,
        ],
        compiler_params=pltpu.CompilerParams(needs_layout_passes=False),
    )(_hist_body)


def _finalize_body(part_ref, out_ref):
    s = part_ref[0] + part_ref[1]
    col = jnp.sum(s[:, 0:LANES], axis=1, keepdims=True)
    row = jnp.sum(s[:, LANES:2 * LANES], axis=1, keepdims=True)
    diag = jnp.sum(s[:, 2 * LANES:3 * LANES], axis=1, keepdims=True)
    total = jnp.float32(N_ROWS)
    tn = total - row - col + diag
    fp = row - diag
    ratio = tn / (tn + fp + jnp.float32(EPS_SPEC))
    classid = lax.broadcasted_iota(jnp.int32, (CPAD, 1), 0)
    ratio = jnp.where(classid < N_CLS, ratio, 0.0)
    out_ref[...] = lax.broadcast(jnp.sum(ratio) / N_CLS, (1, 1))


_finalize_call = pl.pallas_call(
    _finalize_body,
    out_shape=jax.ShapeDtypeStruct((1, 1), jnp.float32),
)


@jax.jit
def kernel(y_pred, y_true):
    preds = _argmax_call(y_pred.T).reshape(N_ROWS)
    partials = _hist_call()(preds, y_true)
    return _finalize_call(partials)[0, 0]


# speedup vs baseline: 6.8234x; 1.0101x over previous
import functools

import jax
import jax.numpy as jnp
from jax import lax
from jax.experimental import pallas as pl
from jax.experimental.pallas import tpu as pltpu
from jax.experimental.pallas import tpu_sc as plsc

N_ROWS = 524288
N_CLS = 100
EPS_SPEC = 1e-07

BLK = 32768
GRID = N_ROWS // BLK

NC, NS, LANES = 2, 16, 16
NW = NC * NS
CHUNK = N_ROWS // NW
NVEC = CHUNK // LANES
CPAD = 128
HSECT = LANES * CPAD
HWORDS = 3 * HSECT


def _argmax_body(ypt_ref, out_ref):
    yp = ypt_ref[...]
    m = jnp.max(yp, axis=0, keepdims=True)
    rowid_f = lax.broadcasted_iota(jnp.int32, (N_CLS, BLK), 0).astype(
        jnp.float32)
    cand = jnp.where(yp == m, rowid_f, jnp.float32(N_CLS))
    pred = jnp.min(cand, axis=0, keepdims=True)
    out_ref[...] = pred.astype(jnp.int32)


_argmax_call = pl.pallas_call(
    _argmax_body,
    grid=(GRID,),
    in_specs=[pl.BlockSpec((N_CLS, BLK), lambda i: (0, i))],
    out_specs=pl.BlockSpec((1, BLK), lambda i: (0, i)),
    out_shape=jax.ShapeDtypeStruct((1, N_ROWS), jnp.int32),
)


def _hist_body(preds_hbm, yt_hbm, out_hbm, pv, tv, hist, idxv, shared, sem):
    cid = lax.axis_index("c")
    sid = lax.axis_index("s")
    wid = cid * NS + sid
    base = wid * CHUNK

    pltpu.sync_copy(preds_hbm.at[pl.ds(base, CHUNK)], pv)
    pltpu.sync_copy(yt_hbm.at[pl.ds(base, CHUNK)], tv)

    zeros16 = jnp.zeros((LANES,), jnp.float32)

    def zero_body(j, _):
        for k in range(CPAD // LANES):
            hist[j, pl.ds(k * LANES, LANES)] = zeros16
        return 0

    lax.fori_loop(0, CPAD, zero_body, 0)

    lane = jnp.arange(LANES, dtype=jnp.int32)
    ones16 = jnp.ones((LANES,), jnp.float32)

    UNROLL = 8

    def acc_body(j, _):
        for u in range(UNROLL):
            p = pv[pl.ds((j * UNROLL + u) * LANES, LANES)]
            t = tv[pl.ds((j * UNROLL + u) * LANES, LANES)]
            plsc.addupdate_scatter(hist, [p, lane], ones16)
            plsc.addupdate_scatter(hist, [t, lane + LANES], ones16)
            hit = jnp.where(p == t, 1.0, 0.0).astype(jnp.float32)
            plsc.addupdate_scatter(hist, [p, lane + 2 * LANES], hit)
        return 0

    lax.fori_loop(0, NVEC // UNROLL, acc_body, 0)

    def idx_body(j, _):
        idxv[pl.ds(j * LANES, LANES)] = lane + j * LANES
        return 0

    lax.fori_loop(0, CPAD // LANES, idx_body, 0)

    @pl.when(sid == 0)
    def _seed():
        pltpu.sync_copy(hist, shared)

    plsc.subcore_barrier()

    @pl.when(sid != 0)
    def _accum():
        pltpu.sync_copy(hist, shared.at[idxv], add=True)

    plsc.subcore_barrier()

    @pl.when(sid == 0)
    def _emit():
        pltpu.sync_copy(shared, out_hbm.at[cid])


@functools.cache
def _hist_call():
    return functools.partial(
        pl.kernel,
        out_type=jax.ShapeDtypeStruct((NC, CPAD, CPAD), jnp.float32),
        mesh=plsc.VectorSubcoreMesh(core_axis_name="c", subcore_axis_name="s",
                                    num_cores=NC, num_subcores=NS),
        scratch_types=[
            pltpu.VMEM((CHUNK,), jnp.int32),
            pltpu.VMEM((CHUNK,), jnp.int32),
            pltpu.VMEM((CPAD, CPAD), jnp.float32),
            pltpu.VMEM((CPAD,), jnp.int32),
            pltpu.VMEM_SHARED((CPAD, CPAD), jnp.float32),
            pltpu.SemaphoreType.DMA,
        ],
        compiler_params=pltpu.CompilerParams(needs_layout_passes=False),
    )(_hist_body)


def _finalize_body(part_ref, out_ref):
    s = part_ref[0] + part_ref[1]
    col = jnp.sum(s[:, 0:LANES], axis=1, keepdims=True)
    row = jnp.sum(s[:, LANES:2 * LANES], axis=1, keepdims=True)
    diag = jnp.sum(s[:, 2 * LANES:3 * LANES], axis=1, keepdims=True)
    total = jnp.float32(N_ROWS)
    tn = total - row - col + diag
    fp = row - diag
    ratio = tn / (tn + fp + jnp.float32(EPS_SPEC))
    classid = lax.broadcasted_iota(jnp.int32, (CPAD, 1), 0)
    ratio = jnp.where(classid < N_CLS, ratio, 0.0)
    out_ref[...] = lax.broadcast(jnp.sum(ratio) / N_CLS, (1, 1))


_finalize_call = pl.pallas_call(
    _finalize_body,
    out_shape=jax.ShapeDtypeStruct((1, 1), jnp.float32),
)


@jax.jit
def kernel(y_pred, y_true):
    preds = _argmax_call(y_pred.T).reshape(N_ROWS)
    partials = _hist_call()(preds, y_true)
    return _finalize_call(partials)[0, 0]


# speedup vs baseline: 6.8271x; 1.0005x over previous
import functools

import jax
import jax.numpy as jnp
from jax import lax
from jax.experimental import pallas as pl
from jax.experimental.pallas import tpu as pltpu
from jax.experimental.pallas import tpu_sc as plsc

N_ROWS = 524288
N_CLS = 100
EPS_SPEC = 1e-07

BLK = 32768
GRID = N_ROWS // BLK

NC, NS, LANES = 2, 16, 16
NW = NC * NS
CHUNK = N_ROWS // NW
NVEC = CHUNK // LANES
CPAD = 128
HSECT = LANES * CPAD
HWORDS = 3 * HSECT


def _argmax_body(ypt_ref, out_ref):
    yp = ypt_ref[...]
    m = jnp.max(yp, axis=0, keepdims=True)
    rowid_f = lax.broadcasted_iota(jnp.int32, (N_CLS, BLK), 0).astype(
        jnp.float32)
    cand = jnp.where(yp == m, rowid_f, jnp.float32(N_CLS))
    pred = jnp.min(cand, axis=0, keepdims=True)
    out_ref[...] = pred.astype(jnp.int32)


_argmax_call = pl.pallas_call(
    _argmax_body,
    grid=(GRID,),
    in_specs=[pl.BlockSpec((N_CLS, BLK), lambda i: (0, i))],
    out_specs=pl.BlockSpec((1, BLK), lambda i: (0, i)),
    out_shape=jax.ShapeDtypeStruct((1, N_ROWS), jnp.int32),
)


def _hist_body(preds_hbm, yt_hbm, out_hbm, pv, tv, hist, idxv, shared, sem):
    cid = lax.axis_index("c")
    sid = lax.axis_index("s")
    wid = cid * NS + sid
    base = wid * CHUNK

    pltpu.sync_copy(preds_hbm.at[pl.ds(base, CHUNK)], pv)
    pltpu.sync_copy(yt_hbm.at[pl.ds(base, CHUNK)], tv)

    zeros16 = jnp.zeros((LANES,), jnp.int32)

    def zero_body(j, _):
        hist[j, pl.ds(0, LANES)] = zeros16
        hist[j, pl.ds(LANES, LANES)] = zeros16
        return 0

    lax.fori_loop(0, CPAD, zero_body, 0)

    lane = jnp.arange(LANES, dtype=jnp.int32)
    ones16 = jnp.ones((LANES,), jnp.int32)
    enc_hit = jnp.full((LANES,), 1 + (1 << 15), jnp.int32)

    UNROLL = 8

    def acc_body(j, _):
        for u in range(UNROLL):
            p = pv[pl.ds((j * UNROLL + u) * LANES, LANES)]
            t = tv[pl.ds((j * UNROLL + u) * LANES, LANES)]
            enc = jnp.where(p == t, enc_hit, ones16)
            plsc.addupdate_scatter(hist, [p, lane], enc)
            plsc.addupdate_scatter(hist, [t, lane + LANES], ones16)
        return 0

    lax.fori_loop(0, NVEC // UNROLL, acc_body, 0)

    def dec_body(j, _):
        ep = hist[j, pl.ds(0, LANES)]
        hist[j, pl.ds(0, LANES)] = ep & 0x7FFF
        hist[j, pl.ds(2 * LANES, LANES)] = lax.shift_right_logical(ep, 15)
        return 0

    lax.fori_loop(0, CPAD, dec_body, 0)

    def idx_body(j, _):
        idxv[pl.ds(j * LANES, LANES)] = lane + j * LANES
        return 0

    lax.fori_loop(0, CPAD // LANES, idx_body, 0)

    @pl.when(sid == 0)
    def _seed():
        pltpu.sync_copy(hist, shared)

    plsc.subcore_barrier()

    @pl.when(sid != 0)
    def _accum():
        pltpu.sync_copy(hist, shared.at[idxv], add=True)

    plsc.subcore_barrier()

    @pl.when(sid == 0)
    def _emit():
        pltpu.sync_copy(shared, out_hbm.at[cid])


@functools.cache
def _hist_call():
    return functools.partial(
        pl.kernel,
        out_type=jax.ShapeDtypeStruct((NC, CPAD, CPAD), jnp.int32),
        mesh=plsc.VectorSubcoreMesh(core_axis_name="c", subcore_axis_name="s",
                                    num_cores=NC, num_subcores=NS),
        scratch_types=[
            pltpu.VMEM((CHUNK,), jnp.int32),
            pltpu.VMEM((CHUNK,), jnp.int32),
            pltpu.VMEM((CPAD, CPAD), jnp.int32),
            pltpu.VMEM((CPAD,), jnp.int32),
            pltpu.VMEM_SHARED((CPAD, CPAD), jnp.int32),
            pltpu.SemaphoreType.DMA,
        ],
        compiler_params=pltpu.CompilerParams(needs_layout_passes=False),
    )(_hist_body)


def _finalize_body(part_ref, out_ref):
    s = (part_ref[0] + part_ref[1]).astype(jnp.float32)
    col = jnp.sum(s[:, 0:LANES], axis=1, keepdims=True)
    row = jnp.sum(s[:, LANES:2 * LANES], axis=1, keepdims=True)
    diag = jnp.sum(s[:, 2 * LANES:3 * LANES], axis=1, keepdims=True)
    total = jnp.float32(N_ROWS)
    tn = total - row - col + diag
    fp = row - diag
    ratio = tn / (tn + fp + jnp.float32(EPS_SPEC))
    classid = lax.broadcasted_iota(jnp.int32, (CPAD, 1), 0)
    ratio = jnp.where(classid < N_CLS, ratio, 0.0)
    out_ref[...] = lax.broadcast(jnp.sum(ratio) / N_CLS, (1, 1))


_finalize_call = pl.pallas_call(
    _finalize_body,
    out_shape=jax.ShapeDtypeStruct((1, 1), jnp.float32),
)


@jax.jit
def kernel(y_pred, y_true):
    preds = _argmax_call(y_pred.T).reshape(N_ROWS)
    partials = _hist_call()(preds, y_true)
    return _finalize_call(partials)[0, 0]
